# baseline (speedup 1.0000x reference)
import jax
import jax.numpy as jnp
from jax import lax
from jax.experimental import pallas as pl
from jax.experimental.pallas import tpu as pltpu
from jax.experimental.pallas import tpu_sc as plsc

B, L, D = 16, 2048, 768
N = B * L
V = 1024

N_SC = 7168
N_TC = N - N_SC

NUM_CORES = 2
NUM_SUBCORES = 16
HD = D // NUM_CORES
TPW = N_SC // NUM_SUBCORES
C = 16
NCHUNK = TPW // C
NVEC = HD // 16

TB = 512
GRID = N_TC // TB


def _sc_body(x0s, y0s, x1s, y1s,
             x0_t, y0_t, x1_t, y1_t, w_t, h_t,
             out_hbm,
             x0_v, y0_v, x1_v, y1_v,
             b00, b01, b02, b03, b04, b05,
             b10, b11, b12, b13, b14, b15,
             sem_g, sem_o):
    hid = lax.axis_index("c")
    tid = lax.axis_index("s")
    base = tid * TPW
    hoff = hid * HD
    tables = (x0_t, y0_t, x1_t, y1_t, w_t, h_t)
    bufs = ((b00, b01, b02, b03, b04, b05),
            (b10, b11, b12, b13, b14, b15))

    pltpu.sync_copy(x0s.at[pl.ds(base, TPW)], x0_v)
    pltpu.sync_copy(y0s.at[pl.ds(base, TPW)], y0_v)
    pltpu.sync_copy(x1s.at[pl.ds(base, TPW)], x1_v)
    pltpu.sync_copy(y1s.at[pl.ds(base, TPW)], y1_v)

    def indices(i):
        off = i * C
        x0 = x0_v[pl.ds(off, C)]
        y0 = y0_v[pl.ds(off, C)]
        x1 = x1_v[pl.ds(off, C)]
        y1 = y1_v[pl.ds(off, C)]
        zero = jnp.zeros((C,), jnp.int32)
        hi = jnp.full((C,), V - 1, jnp.int32)
        x0c = jnp.minimum(jnp.maximum(x0, zero), hi)
        y0c = jnp.minimum(jnp.maximum(y0, zero), hi)
        x1c = jnp.minimum(jnp.maximum(x1, zero), hi)
        y1c = jnp.minimum(jnp.maximum(y1, zero), hi)
        wc = jnp.minimum(jnp.maximum(x1c - x0c, zero), hi)
        hc = jnp.minimum(jnp.maximum(y1c - y0c, zero), hi)
        return (x0c, y0c, x1c, y1c, wc, hc)

    def fire(i, slot):
        idx = indices(i)
        for t in range(6):
            pltpu.async_copy(tables[t].at[idx[t], pl.ds(hoff, HD)],
                             bufs[slot][t], sem_g)

    def wait_gathers(i, slot):
        idx = indices(i)
        for t in range(6):
            pltpu.make_async_copy(tables[t].at[idx[t], pl.ds(hoff, HD)],
                                  bufs[slot][t], sem_g).wait()

    def out_slice(i):
        return out_hbm.at[pl.ds(base + i * C, C), pl.ds(hoff, HD)]

    def sum_and_store(i, slot):
        sb = bufs[slot]

        def jstep(j, _):
            for v in range(NVEC):
                sl = pl.ds(v * 16, 16)
                acc = ((sb[0][j, sl] + sb[1][j, sl])
                       + (sb[2][j, sl] + sb[3][j, sl])
                       + (sb[4][j, sl] + sb[5][j, sl]))
                sb[0][j, sl] = acc
            return 0

        lax.fori_loop(0, C, jstep, 0)
        pltpu.async_copy(sb[0], out_slice(i), sem_o)

    def wait_out(i, slot):
        pltpu.make_async_copy(bufs[slot][0], out_slice(i), sem_o).wait()

    fire(0, 0)

    def step2(g, _):
        for s in range(2):
            i = g * 2 + s
            ns = 1 - s

            @pl.when(i + 1 < NCHUNK)
            def _():
                @pl.when(i >= 1)
                def _():
                    wait_out(i - 1, ns)
                fire(i + 1, ns)

            wait_gathers(i, s)
            sum_and_store(i, s)
        return 0

    lax.fori_loop(0, NCHUNK // 2, step2, 0)
    wait_out(NCHUNK - 2, 0)
    wait_out(NCHUNK - 1, 1)


def _sc_run(x0s, y0s, x1s, y1s, x0_t, y0_t, x1_t, y1_t, w_t, h_t):
    mesh = plsc.VectorSubcoreMesh(
        core_axis_name="c", subcore_axis_name="s",
        num_cores=NUM_CORES, num_subcores=NUM_SUBCORES)
    buf = pltpu.VMEM((C, HD), jnp.float32)
    f = pl.kernel(
        _sc_body,
        out_type=jax.ShapeDtypeStruct((N_SC, D), jnp.float32),
        mesh=mesh,
        scratch_types=[
            pltpu.VMEM((TPW,), jnp.int32),
            pltpu.VMEM((TPW,), jnp.int32),
            pltpu.VMEM((TPW,), jnp.int32),
            pltpu.VMEM((TPW,), jnp.int32),
            buf, buf, buf, buf, buf, buf,
            buf, buf, buf, buf, buf, buf,
            pltpu.SemaphoreType.DMA,
            pltpu.SemaphoreType.DMA,
        ],
    )
    return f(x0s, y0s, x1s, y1s, x0_t, y0_t, x1_t, y1_t, w_t, h_t)


def _tc_body(x0_r, y0_r, x1_r, y1_r, t0, t1, t2, t3, t4, t5, out_r):
    iota = lax.broadcasted_iota(jnp.int32, (TB, V), 1)
    x0 = jnp.clip(x0_r[0, 0, :], 0, V - 1)
    y0 = jnp.clip(y0_r[0, 0, :], 0, V - 1)
    x1 = jnp.clip(x1_r[0, 0, :], 0, V - 1)
    y1 = jnp.clip(y1_r[0, 0, :], 0, V - 1)
    w = jnp.clip(x1 - x0, 0, V - 1)
    h = jnp.clip(y1 - y0, 0, V - 1)
    acc = jnp.zeros((TB, D), jnp.float32)
    for idx, tab in ((x0, t0), (y0, t1), (x1, t2), (y1, t3), (w, t4), (h, t5)):
        oh = (iota == idx[:, None]).astype(jnp.bfloat16)
        acc = acc + jnp.dot(oh, tab[...],
                            preferred_element_type=jnp.float32)
    out_r[...] = acc


def _tc_run(x0s, y0s, x1s, y1s, t0, t1, t2, t3, t4, t5):
    comp_spec = pl.BlockSpec((1, 1, TB), lambda g: (g, 0, 0))
    tab_spec = pl.BlockSpec((V, D), lambda g: (0, 0))
    return pl.pallas_call(
        _tc_body,
        grid=(GRID,),
        in_specs=[comp_spec] * 4 + [tab_spec] * 6,
        out_specs=pl.BlockSpec((TB, D), lambda g: (g + N_SC // TB, 0)),
        out_shape=jax.ShapeDtypeStruct((N, D), jnp.float32),
        compiler_params=pltpu.CompilerParams(
            dimension_semantics=("arbitrary",)),
    )(x0s, y0s, x1s, y1s, t0, t1, t2, t3, t4, t5)


@jax.jit
def _run(bbox, x0_embed, y0_embed, x1_embed, y1_embed, w_embed, h_embed):
    flat = bbox.reshape(N, 4)
    x0s, y0s, x1s, y1s = (flat[:, k] for k in range(4))

    cast = lambda t: t.astype(jnp.bfloat16)
    comps = [c[N_SC:].reshape(GRID, 1, TB) for c in (x0s, y0s, x1s, y1s)]
    out_tc = _tc_run(*comps,
                     cast(x0_embed), cast(y0_embed), cast(x1_embed),
                     cast(y1_embed), cast(w_embed), cast(h_embed))

    out_sc = _sc_run(x0s[:N_SC], y0s[:N_SC], x1s[:N_SC], y1s[:N_SC],
                     x0_embed, y0_embed, x1_embed, y1_embed,
                     w_embed, h_embed)

    return lax.dynamic_update_slice(out_tc, out_sc, (0, 0)).reshape(B, L, D)


def kernel(bbox, x0_embed, y0_embed, x1_embed, y1_embed, w_embed, h_embed):
    return _run(bbox, x0_embed, y0_embed, x1_embed, y1_embed,
                w_embed, h_embed)


# speedup vs baseline: 1.0205x; 1.0205x over previous
import jax
import jax.numpy as jnp
from jax import lax
from jax.experimental import pallas as pl
from jax.experimental.pallas import tpu as pltpu
from jax.experimental.pallas import tpu_sc as plsc

B, L, D = 16, 2048, 768
N = B * L
V = 1024

N_SC = 8192
N_TC = N - N_SC

NUM_CORES = 2
NUM_SUBCORES = 16
HD = D // NUM_CORES
TPW = N_SC // NUM_SUBCORES
C = 16
NCHUNK = TPW // C
NVEC = HD // 16

TB = 1024
GRID = N_TC // TB


def _sc_body(x0s, y0s, x1s, y1s,
             x0_t, y0_t, x1_t, y1_t, w_t, h_t,
             out_hbm,
             x0_v, y0_v, x1_v, y1_v,
             b00, b01, b02, b03, b04, b05,
             b10, b11, b12, b13, b14, b15,
             sem_g, sem_o):
    hid = lax.axis_index("c")
    tid = lax.axis_index("s")
    base = tid * TPW
    hoff = hid * HD
    tables = (x0_t, y0_t, x1_t, y1_t, w_t, h_t)
    bufs = ((b00, b01, b02, b03, b04, b05),
            (b10, b11, b12, b13, b14, b15))

    pltpu.sync_copy(x0s.at[pl.ds(base, TPW)], x0_v)
    pltpu.sync_copy(y0s.at[pl.ds(base, TPW)], y0_v)
    pltpu.sync_copy(x1s.at[pl.ds(base, TPW)], x1_v)
    pltpu.sync_copy(y1s.at[pl.ds(base, TPW)], y1_v)

    def indices(i):
        off = i * C
        x0 = x0_v[pl.ds(off, C)]
        y0 = y0_v[pl.ds(off, C)]
        x1 = x1_v[pl.ds(off, C)]
        y1 = y1_v[pl.ds(off, C)]
        zero = jnp.zeros((C,), jnp.int32)
        hi = jnp.full((C,), V - 1, jnp.int32)
        x0c = jnp.minimum(jnp.maximum(x0, zero), hi)
        y0c = jnp.minimum(jnp.maximum(y0, zero), hi)
        x1c = jnp.minimum(jnp.maximum(x1, zero), hi)
        y1c = jnp.minimum(jnp.maximum(y1, zero), hi)
        wc = jnp.minimum(jnp.maximum(x1c - x0c, zero), hi)
        hc = jnp.minimum(jnp.maximum(y1c - y0c, zero), hi)
        return (x0c, y0c, x1c, y1c, wc, hc)

    def fire(i, slot):
        idx = indices(i)
        for t in range(6):
            pltpu.async_copy(tables[t].at[idx[t], pl.ds(hoff, HD)],
                             bufs[slot][t], sem_g)

    def wait_gathers(i, slot):
        idx = indices(i)
        for t in range(6):
            pltpu.make_async_copy(tables[t].at[idx[t], pl.ds(hoff, HD)],
                                  bufs[slot][t], sem_g).wait()

    def out_slice(i):
        return out_hbm.at[pl.ds(base + i * C, C), pl.ds(hoff, HD)]

    def sum_and_store(i, slot):
        sb = bufs[slot]

        def jstep(j, _):
            for v in range(NVEC):
                sl = pl.ds(v * 16, 16)
                acc = ((sb[0][j, sl] + sb[1][j, sl])
                       + (sb[2][j, sl] + sb[3][j, sl])
                       + (sb[4][j, sl] + sb[5][j, sl]))
                sb[0][j, sl] = acc
            return 0

        lax.fori_loop(0, C, jstep, 0)
        pltpu.async_copy(sb[0], out_slice(i), sem_o)

    def wait_out(i, slot):
        pltpu.make_async_copy(bufs[slot][0], out_slice(i), sem_o).wait()

    fire(0, 0)

    def step2(g, _):
        for s in range(2):
            i = g * 2 + s
            ns = 1 - s

            @pl.when(i + 1 < NCHUNK)
            def _():
                @pl.when(i >= 1)
                def _():
                    wait_out(i - 1, ns)
                fire(i + 1, ns)

            wait_gathers(i, s)
            sum_and_store(i, s)
        return 0

    lax.fori_loop(0, NCHUNK // 2, step2, 0)
    wait_out(NCHUNK - 2, 0)
    wait_out(NCHUNK - 1, 1)


def _sc_run(x0s, y0s, x1s, y1s, x0_t, y0_t, x1_t, y1_t, w_t, h_t):
    mesh = plsc.VectorSubcoreMesh(
        core_axis_name="c", subcore_axis_name="s",
        num_cores=NUM_CORES, num_subcores=NUM_SUBCORES)
    buf = pltpu.VMEM((C, HD), jnp.float32)
    f = pl.kernel(
        _sc_body,
        out_type=jax.ShapeDtypeStruct((N_SC, D), jnp.float32),
        mesh=mesh,
        scratch_types=[
            pltpu.VMEM((TPW,), jnp.int32),
            pltpu.VMEM((TPW,), jnp.int32),
            pltpu.VMEM((TPW,), jnp.int32),
            pltpu.VMEM((TPW,), jnp.int32),
            buf, buf, buf, buf, buf, buf,
            buf, buf, buf, buf, buf, buf,
            pltpu.SemaphoreType.DMA,
            pltpu.SemaphoreType.DMA,
        ],
    )
    return f(x0s, y0s, x1s, y1s, x0_t, y0_t, x1_t, y1_t, w_t, h_t)


def _tc_body(x0_r, y0_r, x1_r, y1_r, t0, t1, t2, t3, t4, t5, out_r):
    iota = lax.broadcasted_iota(jnp.int32, (TB, V), 1)
    x0 = jnp.clip(x0_r[0, 0, :], 0, V - 1)
    y0 = jnp.clip(y0_r[0, 0, :], 0, V - 1)
    x1 = jnp.clip(x1_r[0, 0, :], 0, V - 1)
    y1 = jnp.clip(y1_r[0, 0, :], 0, V - 1)
    w = jnp.clip(x1 - x0, 0, V - 1)
    h = jnp.clip(y1 - y0, 0, V - 1)
    acc = jnp.zeros((TB, D), jnp.float32)
    for idx, tab in ((x0, t0), (y0, t1), (x1, t2), (y1, t3), (w, t4), (h, t5)):
        oh = (iota == idx[:, None]).astype(jnp.bfloat16)
        acc = acc + jnp.dot(oh, tab[...],
                            preferred_element_type=jnp.float32)
    out_r[...] = acc


def _tc_run(x0s, y0s, x1s, y1s, t0, t1, t2, t3, t4, t5):
    comp_spec = pl.BlockSpec((1, 1, TB), lambda g: (g, 0, 0))
    tab_spec = pl.BlockSpec((V, D), lambda g: (0, 0))
    return pl.pallas_call(
        _tc_body,
        grid=(GRID,),
        in_specs=[comp_spec] * 4 + [tab_spec] * 6,
        out_specs=pl.BlockSpec((TB, D), lambda g: (g + N_SC // TB, 0)),
        out_shape=jax.ShapeDtypeStruct((N, D), jnp.float32),
        compiler_params=pltpu.CompilerParams(
            dimension_semantics=("arbitrary",)),
    )(x0s, y0s, x1s, y1s, t0, t1, t2, t3, t4, t5)


@jax.jit
def _run(bbox, x0_embed, y0_embed, x1_embed, y1_embed, w_embed, h_embed):
    flat = bbox.reshape(N, 4)
    x0s, y0s, x1s, y1s = (flat[:, k] for k in range(4))

    cast = lambda t: t.astype(jnp.bfloat16)
    comps = [c[N_SC:].reshape(GRID, 1, TB) for c in (x0s, y0s, x1s, y1s)]
    out_tc = _tc_run(*comps,
                     cast(x0_embed), cast(y0_embed), cast(x1_embed),
                     cast(y1_embed), cast(w_embed), cast(h_embed))

    out_sc = _sc_run(x0s[:N_SC], y0s[:N_SC], x1s[:N_SC], y1s[:N_SC],
                     x0_embed, y0_embed, x1_embed, y1_embed,
                     w_embed, h_embed)

    return lax.dynamic_update_slice(out_tc, out_sc, (0, 0)).reshape(B, L, D)


def kernel(bbox, x0_embed, y0_embed, x1_embed, y1_embed, w_embed, h_embed):
    return _run(bbox, x0_embed, y0_embed, x1_embed, y1_embed,
                w_embed, h_embed)
